# baseline (speedup 1.0000x reference)
import functools

import jax
import jax.numpy as jnp
from jax import lax
from jax.experimental import pallas as pl
from jax.experimental.pallas import tpu as pltpu
from jax.experimental.pallas import tpu_sc as plsc

B = 4096
T = 200
D_OUT = 112
BT = B * T
NW = 32
L = 128
SPC = 2
CH = L * SPC
N_L = BT // L
N_CH = BT // CH
CH_PER_W = N_CH // NW
SB = B // NW


def _sc_static_gather(sidx0, sidx1, t0, t1):
  mesh = plsc.VectorSubcoreMesh(core_axis_name="c", subcore_axis_name="s")

  @functools.partial(
      pl.kernel,
      out_type=(
          jax.ShapeDtypeStruct((B, 32), jnp.float32),
          jax.ShapeDtypeStruct((B, 32), jnp.float32),
      ),
      mesh=mesh,
      compiler_params=pltpu.CompilerParams(use_tc_tiling_on_sc=False),
      scratch_types=[
          pltpu.VMEM((SB,), jnp.int32),
          pltpu.VMEM((SB, 32), jnp.float32),
          pltpu.SemaphoreType.DMA,
      ],
  )
  def k(s0_hbm, s1_hbm, t0_hbm, t1_hbm, es0_out, es1_out,
        sidx_v, srows_v, sem):
    wid = lax.axis_index("s") * 2 + lax.axis_index("c")
    b0 = wid * SB
    pltpu.sync_copy(s0_hbm.at[pl.ds(b0, SB)], sidx_v)
    pltpu.async_copy(t0_hbm.at[sidx_v], srows_v, sem).wait()
    pltpu.sync_copy(srows_v, es0_out.at[pl.ds(b0, SB)])
    pltpu.sync_copy(s1_hbm.at[pl.ds(b0, SB)], sidx_v)
    pltpu.async_copy(t1_hbm.at[sidx_v], srows_v, sem).wait()
    pltpu.sync_copy(srows_v, es1_out.at[pl.ds(b0, SB)])

  return k(sidx0, sidx1, t0, t1)


def _tc_build_s(es0, es1, sreal):
  def body(s0_ref, s1_ref, sr_ref, out_ref):
    out_ref[...] = jnp.concatenate(
        [s0_ref[...], s1_ref[...], sr_ref[...]], axis=-1)

  return pl.pallas_call(
      body,
      out_shape=jax.ShapeDtypeStruct((B, 72), jnp.float32),
  )(es0, es1, sreal)


def _sc_assemble(didx2d, bidx2d, s_tab, dyn_real2d, dt):
  mesh = plsc.VectorSubcoreMesh(core_axis_name="c", subcore_axis_name="s")

  @functools.partial(
      pl.kernel,
      out_type=jax.ShapeDtypeStruct((BT, D_OUT), jnp.float32),
      mesh=mesh,
      compiler_params=pltpu.CompilerParams(use_tc_tiling_on_sc=False),
      scratch_types=[
          pltpu.VMEM((2, SPC, L), jnp.int32),
          pltpu.VMEM((2, SPC, L), jnp.int32),
          pltpu.VMEM((2, CH, 32), jnp.float32),
          pltpu.VMEM((2, CH, 72), jnp.float32),
          pltpu.VMEM((2, CH, 8), jnp.float32),
          pltpu.SemaphoreType.DMA,
          pltpu.SemaphoreType.DMA,
          pltpu.SemaphoreType.DMA,
          pltpu.SemaphoreType.DMA,
      ],
  )
  def k(didx_hbm, bidx_hbm, s_hbm, dr_hbm, dt_hbm, out_hbm,
        didx_v, bidx_v, drows_v, srows_v, dreal_v,
        sem_st, sem_g, sem_wr, sem_wd):
    cid = lax.axis_index("c")
    sid = lax.axis_index("s")
    wid = sid * 2 + cid
    c0 = wid * CH_PER_W

    def a_pairs(g, s):
      q0 = (c0 + g) * SPC
      r0 = (c0 + g) * CH
      return [
          (didx_hbm.at[pl.ds(q0, SPC)], didx_v.at[s]),
          (bidx_hbm.at[pl.ds(q0, SPC)], bidx_v.at[s]),
          (dr_hbm.at[pl.ds(r0, CH)], dreal_v.at[s]),
      ]

    def b_pairs(s):
      ps = []
      for j in range(SPC):
        dst = pl.ds(j * L, L)
        ps.append((dt_hbm.at[didx_v.at[s, j]], drows_v.at[s, dst]))
        ps.append((s_hbm.at[bidx_v.at[s, j]], srows_v.at[s, dst]))
      return ps

    def cr_pairs(g, s):
      r0 = (c0 + g) * CH
      return [
          (srows_v.at[s], out_hbm.at[pl.ds(r0, CH), pl.ds(0, 72)]),
          (drows_v.at[s], out_hbm.at[pl.ds(r0, CH), pl.ds(72, 32)]),
      ]

    def cd_pairs(g, s):
      r0 = (c0 + g) * CH
      return [
          (dreal_v.at[s], out_hbm.at[pl.ds(r0, CH), pl.ds(104, 8)]),
      ]

    def issue(pairs, sem):
      for src, dst in pairs:
        pltpu.async_copy(src, dst, sem)

    def drain(pairs, sem):
      for src, dst in pairs:
        pltpu.make_async_copy(src, dst, sem).wait()

    def step(g, s, first):
      drain(a_pairs(g, s), sem_st)
      if not first:
        drain(cr_pairs(g - 2, s), sem_wr)
      issue(b_pairs(s), sem_g)
      drain(cd_pairs(g - 1, 1 - s), sem_wd)
      issue(a_pairs(g + 1, 1 - s), sem_st)
      drain(b_pairs(s), sem_g)
      issue(cr_pairs(g, s), sem_wr)
      issue(cd_pairs(g, s), sem_wd)

    issue(a_pairs(0, 0), sem_st)
    drain(a_pairs(0, 0), sem_st)
    issue(b_pairs(0), sem_g)
    issue(a_pairs(1, 1), sem_st)
    drain(b_pairs(0), sem_g)
    issue(cr_pairs(0, 0), sem_wr)
    issue(cd_pairs(0, 0), sem_wd)

    step(1, 1, True)
    step(2, 0, False)

    def body(p, carry):
      g = 3 + 2 * p
      step(g, 1, False)
      step(g + 1, 0, False)
      return carry

    lax.fori_loop(0, (CH_PER_W - 4) // 2, body, 0)

    g_last = CH_PER_W - 1
    drain(a_pairs(g_last, 1), sem_st)
    drain(cr_pairs(g_last - 2, 1), sem_wr)
    issue(b_pairs(1), sem_g)
    drain(cd_pairs(g_last - 1, 0), sem_wd)
    drain(b_pairs(1), sem_g)
    issue(cr_pairs(g_last, 1), sem_wr)
    issue(cd_pairs(g_last, 1), sem_wd)
    drain(cr_pairs(g_last - 1, 0), sem_wr)
    drain(cr_pairs(g_last, 1), sem_wr)
    drain(cd_pairs(g_last, 1), sem_wd)

  return k(didx2d, bidx2d, s_tab, dyn_real2d, dt)


def kernel(feat_static_cat, feat_static_real, feat_dynamic_cat,
           feat_dynamic_real, static_table0, static_table1, dyn_table0):
  didx2d = feat_dynamic_cat.astype(jnp.int32).reshape(N_L, L)
  bidx2d = jnp.repeat(jnp.arange(B, dtype=jnp.int32), T).reshape(N_L, L)
  s0 = feat_static_cat[:, 0].astype(jnp.int32)
  s1 = feat_static_cat[:, 1].astype(jnp.int32)
  es0, es1 = _sc_static_gather(s0, s1, static_table0, static_table1)
  s_tab = _tc_build_s(es0, es1, feat_static_real)
  dr2d = feat_dynamic_real.reshape(BT, 8)
  out = _sc_assemble(didx2d, bidx2d, s_tab, dr2d, dyn_table0)
  return out.reshape(B, T, D_OUT)


# speedup vs baseline: 1.2725x; 1.2725x over previous
import functools

import jax
import jax.numpy as jnp
from jax import lax
from jax.experimental import pallas as pl
from jax.experimental.pallas import tpu as pltpu
from jax.experimental.pallas import tpu_sc as plsc

B = 4096
T = 200
D_OUT = 112
BT = B * T
NW = 32
L = 128
SPC = 2
CH = L * SPC
NSLOT = 3
N_L = BT // L
N_CH = BT // CH
CH_PER_W = N_CH // NW


def _sc_assemble(didx2d, s0r2d, s1r2d, bidx2d, sreal, dyn_real2d,
                 t0, t1, dt):
  mesh = plsc.VectorSubcoreMesh(core_axis_name="c", subcore_axis_name="s")

  @functools.partial(
      pl.kernel,
      out_type=jax.ShapeDtypeStruct((BT, D_OUT), jnp.float32),
      mesh=mesh,
      compiler_params=pltpu.CompilerParams(use_tc_tiling_on_sc=False),
      scratch_types=[
          pltpu.VMEM((NSLOT, SPC, L), jnp.int32),
          pltpu.VMEM((NSLOT, SPC, L), jnp.int32),
          pltpu.VMEM((NSLOT, SPC, L), jnp.int32),
          pltpu.VMEM((NSLOT, SPC, L), jnp.int32),
          pltpu.VMEM((NSLOT, CH, 32), jnp.float32),
          pltpu.VMEM((NSLOT, CH, 32), jnp.float32),
          pltpu.VMEM((NSLOT, CH, 32), jnp.float32),
          pltpu.VMEM((NSLOT, CH, 8), jnp.float32),
          pltpu.VMEM((NSLOT, CH, 8), jnp.float32),
          pltpu.SemaphoreType.DMA,
          pltpu.SemaphoreType.DMA,
          pltpu.SemaphoreType.DMA,
          pltpu.SemaphoreType.DMA,
      ],
  )
  def k(didx_hbm, s0r_hbm, s1r_hbm, bidx_hbm, sreal_hbm, dr_hbm,
        t0_hbm, t1_hbm, dt_hbm, out_hbm,
        didx_v, s0i_v, s1i_v, bidx_v, drows_v, s0rows_v, s1rows_v,
        srrows_v, dreal_v, sem_st, sem_g, sem_wr, sem_wd):
    cid = lax.axis_index("c")
    sid = lax.axis_index("s")
    wid = sid * 2 + cid
    c0 = wid * CH_PER_W

    def a_pairs(g, s):
      q0 = (c0 + g) * SPC
      r0 = (c0 + g) * CH
      return [
          (didx_hbm.at[pl.ds(q0, SPC)], didx_v.at[s]),
          (s0r_hbm.at[pl.ds(q0, SPC)], s0i_v.at[s]),
          (s1r_hbm.at[pl.ds(q0, SPC)], s1i_v.at[s]),
          (bidx_hbm.at[pl.ds(q0, SPC)], bidx_v.at[s]),
          (dr_hbm.at[pl.ds(r0, CH)], dreal_v.at[s]),
      ]

    def b_pairs(s):
      ps = []
      for j in range(SPC):
        dst = pl.ds(j * L, L)
        ps.append((dt_hbm.at[didx_v.at[s, j]], drows_v.at[s, dst]))
        ps.append((t0_hbm.at[s0i_v.at[s, j]], s0rows_v.at[s, dst]))
        ps.append((t1_hbm.at[s1i_v.at[s, j]], s1rows_v.at[s, dst]))
        ps.append((sreal_hbm.at[bidx_v.at[s, j]], srrows_v.at[s, dst]))
      return ps

    def cr_pairs(g, s):
      r0 = (c0 + g) * CH
      return [
          (s0rows_v.at[s], out_hbm.at[pl.ds(r0, CH), pl.ds(0, 32)]),
          (s1rows_v.at[s], out_hbm.at[pl.ds(r0, CH), pl.ds(32, 32)]),
          (srrows_v.at[s], out_hbm.at[pl.ds(r0, CH), pl.ds(64, 8)]),
          (drows_v.at[s], out_hbm.at[pl.ds(r0, CH), pl.ds(72, 32)]),
      ]

    def cd_pairs(g, s):
      r0 = (c0 + g) * CH
      return [
          (dreal_v.at[s], out_hbm.at[pl.ds(r0, CH), pl.ds(104, 8)]),
      ]

    def issue(pairs, sem):
      for src, dst in pairs:
        pltpu.async_copy(src, dst, sem)

    def drain(pairs, sem):
      for src, dst in pairs:
        pltpu.make_async_copy(src, dst, sem).wait()

    def slot(g):
      return g % NSLOT

    def step(g, full):
      drain(a_pairs(g + 1, slot(g + 1)), sem_st)
      if full:
        drain(cr_pairs(g - 2, slot(g - 2)), sem_wr)
      issue(b_pairs(slot(g + 1)), sem_g)
      drain(cd_pairs(g - 1, slot(g - 1)), sem_wd)
      issue(a_pairs(g + 2, slot(g + 2)), sem_st)
      drain(b_pairs(slot(g)), sem_g)
      issue(cr_pairs(g, slot(g)), sem_wr)
      issue(cd_pairs(g, slot(g)), sem_wd)

    issue(a_pairs(0, 0), sem_st)
    drain(a_pairs(0, 0), sem_st)
    issue(b_pairs(0), sem_g)
    issue(a_pairs(1, 1), sem_st)
    drain(a_pairs(1, 1), sem_st)
    issue(b_pairs(1), sem_g)
    issue(a_pairs(2, 2), sem_st)
    drain(b_pairs(0), sem_g)
    issue(cr_pairs(0, 0), sem_wr)
    issue(cd_pairs(0, 0), sem_wd)

    def step1(g):
      drain(a_pairs(g + 1, slot(g + 1)), sem_st)
      issue(b_pairs(slot(g + 1)), sem_g)
      drain(cd_pairs(g - 1, slot(g - 1)), sem_wd)
      issue(a_pairs(g + 2, slot(g + 2)), sem_st)
      drain(b_pairs(slot(g)), sem_g)
      issue(cr_pairs(g, slot(g)), sem_wr)
      issue(cd_pairs(g, slot(g)), sem_wd)

    step1(1)

    def body(p, carry):
      g = 2 + 3 * p
      step(g, True)
      step(g + 1, True)
      step(g + 2, True)
      return carry

    lax.fori_loop(0, (CH_PER_W - 4) // 3, body, 0)

    g98, g99 = CH_PER_W - 2, CH_PER_W - 1
    drain(a_pairs(g99, slot(g99)), sem_st)
    drain(cr_pairs(g98 - 2, slot(g98 - 2)), sem_wr)
    issue(b_pairs(slot(g99)), sem_g)
    drain(cd_pairs(g98 - 1, slot(g98 - 1)), sem_wd)
    drain(b_pairs(slot(g98)), sem_g)
    issue(cr_pairs(g98, slot(g98)), sem_wr)
    issue(cd_pairs(g98, slot(g98)), sem_wd)
    drain(cr_pairs(g99 - 2, slot(g99 - 2)), sem_wr)
    drain(b_pairs(slot(g99)), sem_g)
    issue(cr_pairs(g99, slot(g99)), sem_wr)
    issue(cd_pairs(g99, slot(g99)), sem_wd)
    drain(cr_pairs(g98, slot(g98)), sem_wr)
    drain(cr_pairs(g99, slot(g99)), sem_wr)
    drain(cd_pairs(g98, slot(g98)), sem_wd)
    drain(cd_pairs(g99, slot(g99)), sem_wd)

  return k(didx2d, s0r2d, s1r2d, bidx2d, sreal, dyn_real2d, t0, t1, dt)


def kernel(feat_static_cat, feat_static_real, feat_dynamic_cat,
           feat_dynamic_real, static_table0, static_table1, dyn_table0):
  didx2d = feat_dynamic_cat.astype(jnp.int32).reshape(N_L, L)
  s0r2d = jnp.repeat(feat_static_cat[:, 0].astype(jnp.int32),
                     T).reshape(N_L, L)
  s1r2d = jnp.repeat(feat_static_cat[:, 1].astype(jnp.int32),
                     T).reshape(N_L, L)
  bidx2d = jnp.repeat(jnp.arange(B, dtype=jnp.int32), T).reshape(N_L, L)
  dr2d = feat_dynamic_real.reshape(BT, 8)
  out = _sc_assemble(didx2d, s0r2d, s1r2d, bidx2d, feat_static_real, dr2d,
                     static_table0, static_table1, dyn_table0)
  return out.reshape(B, T, D_OUT)


# speedup vs baseline: 1.7777x; 1.3969x over previous
import functools

import jax
import jax.numpy as jnp
from jax import lax
from jax.experimental import pallas as pl
from jax.experimental.pallas import tpu as pltpu
from jax.experimental.pallas import tpu_sc as plsc

B = 4096
T = 200
D_OUT = 112
BT = B * T
NW = 32
CH = 128
G = 8
N_CH = BT // CH
CH_PER_W = N_CH // NW
NG = CH_PER_W // G
SB = B // NW
BB = 8


def _sc_gather(idx2d, sidx0, sidx1, dyn_table, st0, st1):
  mesh = plsc.VectorSubcoreMesh(core_axis_name="c", subcore_axis_name="s")

  @functools.partial(
      pl.kernel,
      out_type=(
          jax.ShapeDtypeStruct((BT, 32), jnp.float32),
          jax.ShapeDtypeStruct((B, 32), jnp.float32),
          jax.ShapeDtypeStruct((B, 32), jnp.float32),
      ),
      mesh=mesh,
      compiler_params=pltpu.CompilerParams(use_tc_tiling_on_sc=False),
      scratch_types=[
          pltpu.VMEM((G, CH), jnp.int32),
          pltpu.VMEM((G * CH, 32), jnp.float32),
          pltpu.VMEM((SB,), jnp.int32),
          pltpu.VMEM((SB, 32), jnp.float32),
          pltpu.SemaphoreType.DMA,
      ],
  )
  def k(idx_hbm, s0_hbm, s1_hbm, tbl_hbm, t0_hbm, t1_hbm,
        dyn_out, es0_out, es1_out, idx_v, rows_v, sidx_v, srows_v, sem):
    wid = lax.axis_index("s") * 2 + lax.axis_index("c")
    c0 = wid * CH_PER_W

    def group(g, carry):
      pltpu.sync_copy(idx_hbm.at[pl.ds(c0 + g * G, G)], idx_v)
      cps = [
          pltpu.async_copy(tbl_hbm.at[idx_v.at[j]],
                           rows_v.at[pl.ds(j * CH, CH)], sem)
          for j in range(G)
      ]
      for cp in cps:
        cp.wait()
      pltpu.sync_copy(rows_v, dyn_out.at[pl.ds((c0 + g * G) * CH, G * CH)])
      return carry

    lax.fori_loop(0, NG, group, 0)

    b0 = wid * SB
    pltpu.sync_copy(s0_hbm.at[pl.ds(b0, SB)], sidx_v)
    pltpu.async_copy(t0_hbm.at[sidx_v], srows_v, sem).wait()
    pltpu.sync_copy(srows_v, es0_out.at[pl.ds(b0, SB)])
    pltpu.sync_copy(s1_hbm.at[pl.ds(b0, SB)], sidx_v)
    pltpu.async_copy(t1_hbm.at[sidx_v], srows_v, sem).wait()
    pltpu.sync_copy(srows_v, es1_out.at[pl.ds(b0, SB)])

  return k(idx2d, sidx0, sidx1, dyn_table, st0, st1)


def _tc_assemble(es0, es1, sreal, emb_dyn, dyn_real_t):
  def body(s0_ref, s1_ref, sr_ref, ed_ref, drt_ref, out_ref):
    stat = jnp.concatenate([s0_ref[...], s1_ref[...], sr_ref[...]], axis=-1)
    statb = jnp.broadcast_to(stat[:, :, None], (BB, 72, T))
    ed = ed_ref[...].reshape(BB, T, 32)
    edt = jnp.transpose(ed, (0, 2, 1))
    out_ref[...] = jnp.concatenate([statb, edt, drt_ref[...]], axis=1)

  return pl.pallas_call(
      body,
      grid=(B // BB,),
      out_shape=jax.ShapeDtypeStruct((B, D_OUT, T), jnp.float32),
      in_specs=[
          pl.BlockSpec((BB, 32), lambda i: (i, 0)),
          pl.BlockSpec((BB, 32), lambda i: (i, 0)),
          pl.BlockSpec((BB, 8), lambda i: (i, 0)),
          pl.BlockSpec((BB * T, 32), lambda i: (i, 0)),
          pl.BlockSpec((BB, 8, T), lambda i: (i, 0, 0)),
      ],
      out_specs=pl.BlockSpec((BB, D_OUT, T), lambda i: (i, 0, 0)),
      compiler_params=pltpu.CompilerParams(
          dimension_semantics=("arbitrary",)),
  )(es0, es1, sreal, emb_dyn, dyn_real_t)


def kernel(feat_static_cat, feat_static_real, feat_dynamic_cat,
           feat_dynamic_real, static_table0, static_table1, dyn_table0):
  idx2d = feat_dynamic_cat.astype(jnp.int32).reshape(N_CH, CH)
  s0 = feat_static_cat[:, 0].astype(jnp.int32)
  s1 = feat_static_cat[:, 1].astype(jnp.int32)
  emb_dyn, es0, es1 = _sc_gather(idx2d, s0, s1, dyn_table0,
                                 static_table0, static_table1)
  drt = jnp.swapaxes(feat_dynamic_real, 1, 2)
  out2 = _tc_assemble(es0, es1, feat_static_real, emb_dyn, drt)
  return jnp.swapaxes(out2, 1, 2)
